# baseline (speedup 1.0000x reference)
import functools

import jax
import jax.numpy as jnp
from jax import lax
from jax.experimental import pallas as pl
from jax.experimental.pallas import tpu as pltpu
from jax.experimental.pallas import tpu_sc as plsc

_Q = 4
_RB = 12800


def _sq_pack_body(x_ref, o_ref, p_ref):
    x = x_ref[0]
    o_ref[0] = x * x
    eye = jnp.eye(x.shape[0], dtype=jnp.float32)
    xt = jax.lax.dot_general(
        x, eye,
        dimension_numbers=(((0,), (0,)), ((), ())),
        preferred_element_type=jnp.float32,
        precision=jax.lax.Precision.HIGHEST,
    )
    a = pl.program_id(1)
    for aa in range(_Q):
        @pl.when(a == aa)
        def _():
            p_ref[:, pl.ds(32 * aa, 32)] = xt


@jax.jit
def _square_pack(xt):
    r1, r2, n = xt.shape
    nblk_q = pl.cdiv(pl.cdiv(n, _Q), _RB)
    nqp = nblk_q * _RB
    last = pl.cdiv(n, _RB) - 1

    def in_map(i, a):
        return (0, 0, jnp.minimum(a * nblk_q + i, last))

    return pl.pallas_call(
        _sq_pack_body,
        grid=(nblk_q, _Q),
        in_specs=[pl.BlockSpec((1, r2, _RB), in_map)],
        out_specs=[
            pl.BlockSpec((1, r2, _RB), in_map),
            pl.BlockSpec((_RB, _Q * 32), lambda i, a: (i, 0)),
        ],
        out_shape=[
            jax.ShapeDtypeStruct((r1, r2, n), jnp.float32),
            jax.ShapeDtypeStruct((nqp, _Q * 32), jnp.float32),
        ],
    )(xt)


def _make_gather(NQ, C, B):
    info = plsc.get_sparse_core_info()
    NC, NS = info.num_cores, info.num_subcores
    NW = NC * NS
    CH = 128
    b_per_w = B // NW
    n_ch = b_per_w // CH
    assert b_per_w * NW == B and n_ch * CH == b_per_w

    mesh = plsc.VectorSubcoreMesh(core_axis_name="c", subcore_axis_name="s")

    @functools.partial(
        pl.kernel,
        mesh=mesh,
        compiler_params=pltpu.CompilerParams(
            use_tc_tiling_on_sc=False, needs_layout_passes=False
        ),
        out_type=jax.ShapeDtypeStruct((C, B), jnp.float32),
        scratch_types=[
            pltpu.VMEM((n_ch, CH), jnp.int32),
            pltpu.VMEM((n_ch, CH), jnp.int32),
            pltpu.VMEM((CH, _Q * 32), jnp.float32),
            pltpu.VMEM((C, b_per_w), jnp.float32),
            pltpu.SemaphoreType.DMA,
        ],
    )
    def gather_kernel(packed, idx_hbm, out_hbm, idx_v, rid_v, rows_v, res_v, sem):
        wid = lax.axis_index("s") * NC + lax.axis_index("c")
        pltpu.sync_copy(idx_hbm.at[wid], idx_v)
        for j in range(n_ch):
            for g in range(CH // 16):
                v = idx_v[j, pl.ds(g * 16, 16)]
                a16 = (jnp.where(v >= NQ, 1, 0)
                       + jnp.where(v >= 2 * NQ, 1, 0)
                       + jnp.where(v >= 3 * NQ, 1, 0))
                rid_v[j, pl.ds(g * 16, 16)] = v - NQ * a16
            pltpu.async_copy(packed.at[rid_v.at[j]], rows_v, sem).wait()
            for g in range(CH // 16):
                v = idx_v[j, pl.ds(g * 16, 16)]
                a16 = (jnp.where(v >= NQ, 1, 0)
                       + jnp.where(v >= 2 * NQ, 1, 0)
                       + jnp.where(v >= 3 * NQ, 1, 0))
                colb = 32 * a16
                row16 = lax.iota(jnp.int32, 16) + g * 16
                for c in range(C):
                    vals = plsc.load_gather(rows_v, [row16, colb + c])
                    res_v[c, pl.ds(j * CH + g * 16, 16)] = vals
        pltpu.sync_copy(res_v, out_hbm.at[:, pl.ds(wid * b_per_w, b_per_w)])

    return gather_kernel, NW, n_ch, CH


def kernel(core_param, indices):
    r1, n, r2 = core_param.shape
    b = indices.shape[0]

    xt = jnp.transpose(core_param, (0, 2, 1))
    reg_t, packed = _square_pack(xt)
    reg = jnp.transpose(reg_t, (0, 2, 1))

    gather_fn, nw, n_ch, ch = _make_gather(packed.shape[0], r2, b)
    out_t = gather_fn(packed, indices.reshape(nw, n_ch, ch))
    out = jnp.transpose(out_t.reshape(r1, r2, b), (2, 0, 1))

    return (out, reg)


# speedup vs baseline: 1.8611x; 1.8611x over previous
import functools

import jax
import jax.numpy as jnp
from jax import lax
from jax.experimental import pallas as pl
from jax.experimental.pallas import tpu as pltpu
from jax.experimental.pallas import tpu_sc as plsc

_Q = 4
_RB = 25600


def _sq_pack_body(x_ref, o_ref, p_ref):
    x = x_ref[0]
    o_ref[0] = x * x
    xt = x.T
    a = pl.program_id(1)
    for aa in range(_Q):
        @pl.when(a == aa)
        def _():
            p_ref[:, pl.ds(32 * aa, 32)] = xt


@jax.jit
def _square_pack(xt):
    r1, r2, n = xt.shape
    nblk_q = pl.cdiv(pl.cdiv(n, _Q), _RB)
    nqp = nblk_q * _RB
    last = pl.cdiv(n, _RB) - 1

    def in_map(i, a):
        return (0, 0, jnp.minimum(a * nblk_q + i, last))

    return pl.pallas_call(
        _sq_pack_body,
        grid=(nblk_q, _Q),
        in_specs=[pl.BlockSpec((1, r2, _RB), in_map)],
        out_specs=[
            pl.BlockSpec((1, r2, _RB), in_map),
            pl.BlockSpec((_RB, _Q * 32), lambda i, a: (i, 0)),
        ],
        out_shape=[
            jax.ShapeDtypeStruct((r1, r2, n), jnp.float32),
            jax.ShapeDtypeStruct((nqp, _Q * 32), jnp.float32),
        ],
    )(xt)


def _make_gather(NQ, C, B):
    info = plsc.get_sparse_core_info()
    NC, NS = info.num_cores, info.num_subcores
    NW = NC * NS
    CH = 128
    b_per_w = B // NW
    n_ch = b_per_w // CH
    assert b_per_w * NW == B and n_ch * CH == b_per_w

    mesh = plsc.VectorSubcoreMesh(core_axis_name="c", subcore_axis_name="s")

    @functools.partial(
        pl.kernel,
        mesh=mesh,
        compiler_params=pltpu.CompilerParams(
            use_tc_tiling_on_sc=False, needs_layout_passes=False
        ),
        out_type=jax.ShapeDtypeStruct((C, B), jnp.float32),
        scratch_types=[
            pltpu.VMEM((n_ch, CH), jnp.int32),
            pltpu.VMEM((n_ch, CH), jnp.int32),
            pltpu.VMEM((CH, _Q * 32), jnp.float32),
            pltpu.VMEM((C, b_per_w), jnp.float32),
            pltpu.SemaphoreType.DMA,
        ],
    )
    def gather_kernel(packed, idx_hbm, out_hbm, idx_v, rid_v, rows_v, res_v, sem):
        wid = lax.axis_index("s") * NC + lax.axis_index("c")
        pltpu.sync_copy(idx_hbm.at[wid], idx_v)
        for j in range(n_ch):
            for g in range(CH // 16):
                v = idx_v[j, pl.ds(g * 16, 16)]
                a16 = (jnp.where(v >= NQ, 1, 0)
                       + jnp.where(v >= 2 * NQ, 1, 0)
                       + jnp.where(v >= 3 * NQ, 1, 0))
                rid_v[j, pl.ds(g * 16, 16)] = v - NQ * a16
            pltpu.async_copy(packed.at[rid_v.at[j]], rows_v, sem).wait()
            for g in range(CH // 16):
                v = idx_v[j, pl.ds(g * 16, 16)]
                a16 = (jnp.where(v >= NQ, 1, 0)
                       + jnp.where(v >= 2 * NQ, 1, 0)
                       + jnp.where(v >= 3 * NQ, 1, 0))
                colb = 32 * a16
                row16 = lax.iota(jnp.int32, 16) + g * 16
                for c in range(C):
                    vals = plsc.load_gather(rows_v, [row16, colb + c])
                    res_v[c, pl.ds(j * CH + g * 16, 16)] = vals
        pltpu.sync_copy(res_v, out_hbm.at[:, pl.ds(wid * b_per_w, b_per_w)])

    return gather_kernel, NW, n_ch, CH


def kernel(core_param, indices):
    r1, n, r2 = core_param.shape
    b = indices.shape[0]

    xt = jnp.transpose(core_param, (0, 2, 1))
    reg_t, packed = _square_pack(xt)
    reg = jnp.transpose(reg_t, (0, 2, 1))

    gather_fn, nw, n_ch, ch = _make_gather(packed.shape[0], r2, b)
    out_t = gather_fn(packed, indices.reshape(nw, n_ch, ch))
    out = jnp.transpose(out_t.reshape(r1, r2, b), (2, 0, 1))

    return (out, reg)


# speedup vs baseline: 1.8711x; 1.0053x over previous
import functools

import jax
import jax.numpy as jnp
from jax import lax
from jax.experimental import pallas as pl
from jax.experimental.pallas import tpu as pltpu
from jax.experimental.pallas import tpu_sc as plsc

_Q = 4
_RB = 32000


def _sq_pack_body(x_ref, o_ref, p_ref):
    x = x_ref[0]
    o_ref[0] = x * x
    xt = x.T
    a = pl.program_id(1)
    for aa in range(_Q):
        @pl.when(a == aa)
        def _():
            p_ref[:, pl.ds(32 * aa, 32)] = xt


@jax.jit
def _square_pack(xt):
    r1, r2, n = xt.shape
    nblk_q = pl.cdiv(pl.cdiv(n, _Q), _RB)
    nqp = nblk_q * _RB
    last = pl.cdiv(n, _RB) - 1

    def in_map(i, a):
        return (0, 0, jnp.minimum(a * nblk_q + i, last))

    return pl.pallas_call(
        _sq_pack_body,
        grid=(nblk_q, _Q),
        compiler_params=pltpu.CompilerParams(vmem_limit_bytes=100 * 1024 * 1024),
        in_specs=[pl.BlockSpec((1, r2, _RB), in_map)],
        out_specs=[
            pl.BlockSpec((1, r2, _RB), in_map),
            pl.BlockSpec((_RB, _Q * 32), lambda i, a: (i, 0)),
        ],
        out_shape=[
            jax.ShapeDtypeStruct((r1, r2, n), jnp.float32),
            jax.ShapeDtypeStruct((nqp, _Q * 32), jnp.float32),
        ],
    )(xt)


def _make_gather(NQ, C, B):
    info = plsc.get_sparse_core_info()
    NC, NS = info.num_cores, info.num_subcores
    NW = NC * NS
    CH = 128
    b_per_w = B // NW
    n_ch = b_per_w // CH
    assert b_per_w * NW == B and n_ch * CH == b_per_w

    mesh = plsc.VectorSubcoreMesh(core_axis_name="c", subcore_axis_name="s")

    @functools.partial(
        pl.kernel,
        mesh=mesh,
        compiler_params=pltpu.CompilerParams(
            use_tc_tiling_on_sc=False, needs_layout_passes=False
        ),
        out_type=jax.ShapeDtypeStruct((C, B), jnp.float32),
        scratch_types=[
            pltpu.VMEM((n_ch, CH), jnp.int32),
            pltpu.VMEM((n_ch, CH), jnp.int32),
            pltpu.VMEM((CH, _Q * 32), jnp.float32),
            pltpu.VMEM((C, b_per_w), jnp.float32),
            pltpu.SemaphoreType.DMA,
        ],
    )
    def gather_kernel(packed, idx_hbm, out_hbm, idx_v, rid_v, rows_v, res_v, sem):
        wid = lax.axis_index("s") * NC + lax.axis_index("c")
        pltpu.sync_copy(idx_hbm.at[wid], idx_v)
        for j in range(n_ch):
            for g in range(CH // 16):
                v = idx_v[j, pl.ds(g * 16, 16)]
                a16 = (jnp.where(v >= NQ, 1, 0)
                       + jnp.where(v >= 2 * NQ, 1, 0)
                       + jnp.where(v >= 3 * NQ, 1, 0))
                rid_v[j, pl.ds(g * 16, 16)] = v - NQ * a16
            pltpu.async_copy(packed.at[rid_v.at[j]], rows_v, sem).wait()
            for g in range(CH // 16):
                v = idx_v[j, pl.ds(g * 16, 16)]
                a16 = (jnp.where(v >= NQ, 1, 0)
                       + jnp.where(v >= 2 * NQ, 1, 0)
                       + jnp.where(v >= 3 * NQ, 1, 0))
                colb = 32 * a16
                row16 = lax.iota(jnp.int32, 16) + g * 16
                for c in range(C):
                    vals = plsc.load_gather(rows_v, [row16, colb + c])
                    res_v[c, pl.ds(j * CH + g * 16, 16)] = vals
        pltpu.sync_copy(res_v, out_hbm.at[:, pl.ds(wid * b_per_w, b_per_w)])

    return gather_kernel, NW, n_ch, CH


def kernel(core_param, indices):
    r1, n, r2 = core_param.shape
    b = indices.shape[0]

    xt = jnp.transpose(core_param, (0, 2, 1))
    reg_t, packed = _square_pack(xt)
    reg = jnp.transpose(reg_t, (0, 2, 1))

    gather_fn, nw, n_ch, ch = _make_gather(packed.shape[0], r2, b)
    out_t = gather_fn(packed, indices.reshape(nw, n_ch, ch))
    out = jnp.transpose(out_t.reshape(r1, r2, b), (2, 0, 1))

    return (out, reg)


# speedup vs baseline: 2.9367x; 1.5695x over previous
import functools

import jax
import jax.numpy as jnp
from jax import lax
from jax.experimental import pallas as pl
from jax.experimental.pallas import tpu as pltpu
from jax.experimental.pallas import tpu_sc as plsc


def _sq_body(x_ref, o_ref):
    x = x_ref[...]
    o_ref[...] = x * x


@functools.partial(jax.jit, static_argnums=(1,))
def _square(xt, blk):
    r1, r2, n = xt.shape
    return pl.pallas_call(
        _sq_body,
        grid=(pl.cdiv(n, blk),),
        in_specs=[pl.BlockSpec((r1, r2, blk), lambda i: (0, 0, i))],
        out_specs=pl.BlockSpec((r1, r2, blk), lambda i: (0, 0, i)),
        out_shape=jax.ShapeDtypeStruct((r1, r2, n), jnp.float32),
    )(xt)


def _make_gather(C, B):
    info = plsc.get_sparse_core_info()
    NC, NS = info.num_cores, info.num_subcores
    NW = NC * NS
    G = 16
    b_per_w = B // NW
    n_g = b_per_w // G
    assert b_per_w * NW == B and n_g * G == b_per_w

    mesh = plsc.VectorSubcoreMesh(core_axis_name="c", subcore_axis_name="s")

    @functools.partial(
        pl.kernel,
        mesh=mesh,
        compiler_params=pltpu.CompilerParams(needs_layout_passes=False),
        out_type=jax.ShapeDtypeStruct((C, B), jnp.float32),
        scratch_types=[
            pltpu.VMEM((b_per_w,), jnp.int32),
            pltpu.VMEM((G, C, 128), jnp.float32),
            pltpu.VMEM((C, b_per_w), jnp.float32),
            pltpu.SemaphoreType.DMA,
        ],
    )
    def gather_kernel(tab, idx_hbm, out_hbm, idx_v, ring_v, res_v, sem):
        wid = lax.axis_index("s") * NC + lax.axis_index("c")
        pltpu.sync_copy(idx_hbm.at[pl.ds(wid * b_per_w, b_per_w)], idx_v)
        for g in range(n_g):
            v = idx_v[pl.ds(g * G, G)]
            cols = lax.bitwise_and(v, 127)
            copies = []
            for s in range(G):
                off = pl.multiple_of((v[s] >> 7) * 128, 128)
                copies.append(
                    pltpu.async_copy(
                        tab.at[:, pl.ds(off, 128)],
                        ring_v.at[s],
                        sem,
                    )
                )
            for cp in copies:
                cp.wait()
            slab_ids = lax.iota(jnp.int32, G)
            for c in range(C):
                chan = slab_ids * 0 + c
                vals = plsc.load_gather(ring_v, [slab_ids, chan, cols])
                res_v[c, pl.ds(g * G, G)] = vals
        pltpu.sync_copy(res_v, out_hbm.at[:, pl.ds(wid * b_per_w, b_per_w)])

    return gather_kernel, NW


def kernel(core_param, indices):
    r1, n, r2 = core_param.shape
    b = indices.shape[0]

    xt = jnp.transpose(core_param, (0, 2, 1))
    reg = jnp.transpose(_square(xt, 64000), (0, 2, 1))

    tab = xt.reshape(r2, n)
    gather_fn, nw = _make_gather(r2, b)
    out_t = gather_fn(tab, indices)
    out = jnp.transpose(out_t.reshape(r1, r2, b), (2, 0, 1))

    return (out, reg)
